# initial kernel scaffold (unmeasured)
import jax
import jax.numpy as jnp
from jax import lax
from jax.experimental import pallas as pl
from jax.experimental.pallas import tpu as pltpu

M = 4096
Q = 1024
N = 8192
W = 1024
C = N // W


def kernel(x, dy):
    mx = lax.axis_index("x")
    my = lax.axis_index("y")
    q_mine = 2 * mx + my
    q_send = 2 * (1 - mx) + my

    xb = x.astype(jnp.bfloat16)
    dyb = dy.astype(jnp.bfloat16)
    xs_mine = lax.dynamic_slice(xb, (0, q_mine * Q), (xb.shape[0], Q))
    xs_send = lax.dynamic_slice(xb, (0, q_send * Q), (xb.shape[0], Q))
    p_mine = jnp.dot(
        xs_mine.T, dyb, preferred_element_type=jnp.float32
    ).astype(jnp.bfloat16)
    p_send = jnp.dot(
        xs_send.T, dyb, preferred_element_type=jnp.float32
    ).astype(jnp.bfloat16)

    def body(
        p_mine_ref, p_send_ref, out_ref,
        recv_x, red, recv_y,
        send1, recv1, send2, recv2,
        credit1, credit2,
    ):
        i = pl.program_id(0)
        tx = lax.axis_index("x")
        ty = lax.axis_index("y")
        x_nbr = (1 - tx, ty)
        y_nbr = (tx, 1 - ty)

        barrier_sem = pltpu.get_barrier_semaphore()

        @pl.when(i == 0)
        def _():
            for nbr in (x_nbr, y_nbr):
                pl.semaphore_signal(
                    barrier_sem, inc=1,
                    device_id=nbr, device_id_type=pl.DeviceIdType.MESH,
                )
            pl.semaphore_wait(barrier_sem, 2)

        @pl.when(i > 0)
        def _():
            pl.semaphore_wait(credit1, 1)

        rdma1 = pltpu.make_async_remote_copy(
            src_ref=p_send_ref,
            dst_ref=recv_x,
            send_sem=send1,
            recv_sem=recv1,
            device_id=x_nbr,
            device_id_type=pl.DeviceIdType.MESH,
        )
        rdma1.start()
        rdma1.wait()

        s = p_mine_ref[...].astype(jnp.float32) + recv_x[...].astype(
            jnp.float32
        )
        red[...] = s.astype(jnp.bfloat16)
        pl.semaphore_signal(
            credit1, inc=1,
            device_id=x_nbr, device_id_type=pl.DeviceIdType.MESH,
        )

        @pl.when(i > 0)
        def _():
            pl.semaphore_wait(credit2, 1)

        rdma2 = pltpu.make_async_remote_copy(
            src_ref=red,
            dst_ref=recv_y,
            send_sem=send2,
            recv_sem=recv2,
            device_id=y_nbr,
            device_id_type=pl.DeviceIdType.MESH,
        )
        rdma2.start()
        out_ref[pl.ds(ty * Q, Q), :] = red[...].astype(jnp.float32)
        rdma2.wait()
        out_ref[pl.ds((1 - ty) * Q, Q), :] = recv_y[...].astype(jnp.float32)
        pl.semaphore_signal(
            credit2, inc=1,
            device_id=y_nbr, device_id_type=pl.DeviceIdType.MESH,
        )

    out = pl.pallas_call(
        body,
        grid=(C,),
        out_shape=jax.ShapeDtypeStruct((2 * Q, N), jnp.float32),
        in_specs=[
            pl.BlockSpec((Q, W), lambda i: (0, i)),
            pl.BlockSpec((Q, W), lambda i: (0, i)),
        ],
        out_specs=pl.BlockSpec((2 * Q, W), lambda i: (0, i)),
        scratch_shapes=[
            pltpu.VMEM((Q, W), jnp.bfloat16),
            pltpu.VMEM((Q, W), jnp.bfloat16),
            pltpu.VMEM((Q, W), jnp.bfloat16),
            pltpu.SemaphoreType.DMA,
            pltpu.SemaphoreType.DMA,
            pltpu.SemaphoreType.DMA,
            pltpu.SemaphoreType.DMA,
            pltpu.SemaphoreType.REGULAR,
            pltpu.SemaphoreType.REGULAR,
        ],
        compiler_params=pltpu.CompilerParams(collective_id=0),
    )(p_mine, p_send)
    return out


# baseline (device time: 809629 ns/iter reference)
import jax
import jax.numpy as jnp
from jax import lax
from jax.experimental import pallas as pl
from jax.experimental.pallas import tpu as pltpu

M = 4096
Q = 1024
N = 8192
W = 1024
C = N // W


def kernel(x, dy):
    mx = lax.axis_index("x")
    my = lax.axis_index("y")
    q_mine = 2 * mx + my
    q_send = 2 * (1 - mx) + my

    xb = x.astype(jnp.bfloat16)
    dyb = dy.astype(jnp.bfloat16)
    xs_mine = lax.dynamic_slice(xb, (0, q_mine * Q), (xb.shape[0], Q))
    xs_send = lax.dynamic_slice(xb, (0, q_send * Q), (xb.shape[0], Q))
    p_mine = jnp.dot(
        xs_mine.T, dyb, preferred_element_type=jnp.float32
    ).astype(jnp.bfloat16)
    p_send = jnp.dot(
        xs_send.T, dyb, preferred_element_type=jnp.float32
    ).astype(jnp.bfloat16)

    def body(
        p_mine_ref, p_send_ref, out_ref,
        recv_x, red, recv_y,
        send1, recv1, send2, recv2,
    ):
        i = pl.program_id(0)
        slot = lax.rem(i, 2)
        tx = lax.axis_index("x")
        ty = lax.axis_index("y")
        x_nbr = (1 - tx, ty)
        y_nbr = (tx, 1 - ty)

        barrier_sem = pltpu.get_barrier_semaphore()

        @pl.when(i == 0)
        def _():
            for nbr in (x_nbr, y_nbr):
                pl.semaphore_signal(
                    barrier_sem, inc=1,
                    device_id=nbr, device_id_type=pl.DeviceIdType.MESH,
                )
            pl.semaphore_wait(barrier_sem, 2)

        rdma1 = pltpu.make_async_remote_copy(
            src_ref=p_send_ref,
            dst_ref=recv_x.at[slot],
            send_sem=send1.at[slot],
            recv_sem=recv1.at[slot],
            device_id=x_nbr,
            device_id_type=pl.DeviceIdType.MESH,
        )
        rdma1.start()
        rdma1.wait()

        s = p_mine_ref[...].astype(jnp.float32) + recv_x[slot].astype(
            jnp.float32
        )
        red[...] = s.astype(jnp.bfloat16)

        rdma2 = pltpu.make_async_remote_copy(
            src_ref=red,
            dst_ref=recv_y.at[slot],
            send_sem=send2.at[slot],
            recv_sem=recv2.at[slot],
            device_id=y_nbr,
            device_id_type=pl.DeviceIdType.MESH,
        )
        rdma2.start()
        out_ref[pl.ds(ty * Q, Q), :] = red[...].astype(jnp.float32)
        rdma2.wait()
        out_ref[pl.ds((1 - ty) * Q, Q), :] = recv_y[slot].astype(jnp.float32)

    out = pl.pallas_call(
        body,
        grid=(C,),
        out_shape=jax.ShapeDtypeStruct((2 * Q, N), jnp.float32),
        in_specs=[
            pl.BlockSpec((Q, W), lambda i: (0, i)),
            pl.BlockSpec((Q, W), lambda i: (0, i)),
        ],
        out_specs=pl.BlockSpec((2 * Q, W), lambda i: (0, i)),
        scratch_shapes=[
            pltpu.VMEM((2, Q, W), jnp.bfloat16),
            pltpu.VMEM((Q, W), jnp.bfloat16),
            pltpu.VMEM((2, Q, W), jnp.bfloat16),
            pltpu.SemaphoreType.DMA((2,)),
            pltpu.SemaphoreType.DMA((2,)),
            pltpu.SemaphoreType.DMA((2,)),
            pltpu.SemaphoreType.DMA((2,)),
        ],
        compiler_params=pltpu.CompilerParams(
            collective_id=0, vmem_limit_bytes=56 * 1024 * 1024
        ),
    )(p_mine, p_send)
    return out


# device time: 292823 ns/iter; 2.7649x vs baseline; 2.7649x over previous
import jax
import jax.numpy as jnp
from jax import lax
from jax.experimental import pallas as pl
from jax.experimental.pallas import tpu as pltpu

K = 4096
Q = 1024
N = 8192
W = 512
C = N // W
NSLOT = 3


def kernel(x, dy):
    mx = lax.axis_index("x")
    my = lax.axis_index("y")
    q_mine = 2 * mx + my
    q_send = 2 * (1 - mx) + my

    xs_m = lax.dynamic_slice(x, (0, q_mine * Q), (K, Q))
    xs_s = lax.dynamic_slice(x, (0, q_send * Q), (K, Q))
    xs_mine_t = xs_m.T.astype(jnp.bfloat16)
    xs_send_t = xs_s.T.astype(jnp.bfloat16)

    def body(
        xs_mine_hbm, xs_send_hbm, dy_ref, out_ref,
        xs_mine_vm, xs_send_vm,
        psend, pmine, recvx, red, recvy,
        cp_sems, send1, recv1, send2, recv2,
    ):
        i = pl.program_id(0)
        tx = lax.axis_index("x")
        ty = lax.axis_index("y")
        x_nbr = (1 - tx, ty)
        y_nbr = (tx, 1 - ty)
        barrier_sem = pltpu.get_barrier_semaphore()

        def mk1(slot):
            return pltpu.make_async_remote_copy(
                src_ref=psend.at[slot],
                dst_ref=recvx.at[slot],
                send_sem=send1.at[slot],
                recv_sem=recv1.at[slot],
                device_id=x_nbr,
                device_id_type=pl.DeviceIdType.MESH,
            )

        def mk2(slot):
            return pltpu.make_async_remote_copy(
                src_ref=red.at[slot],
                dst_ref=recvy.at[slot],
                send_sem=send2.at[slot],
                recv_sem=recv2.at[slot],
                device_id=y_nbr,
                device_id_type=pl.DeviceIdType.MESH,
            )

        def store_slot(ref, s, val):
            for k in range(NSLOT):
                @pl.when(s == k)
                def _(k=k):
                    ref[k, :, :] = val

        @pl.when(i == 0)
        def _():
            cp1 = pltpu.make_async_copy(xs_mine_hbm, xs_mine_vm,
                                        cp_sems.at[0])
            cp2 = pltpu.make_async_copy(xs_send_hbm, xs_send_vm,
                                        cp_sems.at[1])
            cp1.start()
            cp2.start()
            for nbr in (x_nbr, y_nbr):
                pl.semaphore_signal(
                    barrier_sem, inc=1,
                    device_id=nbr, device_id_type=pl.DeviceIdType.MESH,
                )
            pl.semaphore_wait(barrier_sem, 2)
            cp1.wait()
            cp2.wait()

        @pl.when(i < C)
        def _():
            s = lax.rem(i, NSLOT)

            @pl.when(i >= NSLOT)
            def _():
                mk1(s).wait_send()

            dyb = dy_ref[...].astype(jnp.bfloat16)
            ps = jnp.dot(
                xs_send_vm[...], dyb, preferred_element_type=jnp.float32
            ).astype(jnp.bfloat16)
            store_slot(psend, s, ps)
            mk1(s).start()
            pm = jnp.dot(
                xs_mine_vm[...], dyb, preferred_element_type=jnp.float32
            ).astype(jnp.bfloat16)
            store_slot(pmine, s, pm)

        @pl.when(jnp.logical_and(i >= 1, i <= C))
        def _():
            j = i - 1
            s = lax.rem(j, NSLOT)
            mk1(s).wait_recv()

            @pl.when(j >= NSLOT)
            def _():
                mk2(s).wait_send()

            rv = (
                pmine[s].astype(jnp.float32) + recvx[s].astype(jnp.float32)
            ).astype(jnp.bfloat16)
            store_slot(red, s, rv)
            mk2(s).start()

        @pl.when(i >= 2)
        def _():
            j = i - 2
            s = lax.rem(j, NSLOT)
            out_ref[pl.ds(ty * Q, Q), :] = red[s].astype(jnp.float32)
            mk2(s).wait_recv()
            out_ref[pl.ds((1 - ty) * Q, Q), :] = recvy[s].astype(
                jnp.float32
            )

        @pl.when(i == C + 1)
        def _():
            for k in range(NSLOT):
                mk1(k).wait_send()
                mk2(k).wait_send()

    out = pl.pallas_call(
        body,
        grid=(C + 2,),
        out_shape=jax.ShapeDtypeStruct((2 * Q, N), jnp.float32),
        in_specs=[
            pl.BlockSpec(memory_space=pl.ANY),
            pl.BlockSpec(memory_space=pl.ANY),
            pl.BlockSpec((K, W), lambda i: (0, jnp.minimum(i, C - 1))),
        ],
        out_specs=pl.BlockSpec(
            (2 * Q, W), lambda i: (0, jnp.maximum(i - 2, 0))
        ),
        scratch_shapes=[
            pltpu.VMEM((Q, K), jnp.bfloat16),
            pltpu.VMEM((Q, K), jnp.bfloat16),
            pltpu.VMEM((NSLOT, Q, W), jnp.bfloat16),
            pltpu.VMEM((NSLOT, Q, W), jnp.bfloat16),
            pltpu.VMEM((NSLOT, Q, W), jnp.bfloat16),
            pltpu.VMEM((NSLOT, Q, W), jnp.bfloat16),
            pltpu.VMEM((NSLOT, Q, W), jnp.bfloat16),
            pltpu.SemaphoreType.DMA((2,)),
            pltpu.SemaphoreType.DMA((NSLOT,)),
            pltpu.SemaphoreType.DMA((NSLOT,)),
            pltpu.SemaphoreType.DMA((NSLOT,)),
            pltpu.SemaphoreType.DMA((NSLOT,)),
        ],
        compiler_params=pltpu.CompilerParams(
            collective_id=0, vmem_limit_bytes=62 * 1024 * 1024
        ),
    )(xs_mine_t, xs_send_t, dy)
    return out


# device time: 278383 ns/iter; 2.9083x vs baseline; 1.0519x over previous
import jax
import jax.numpy as jnp
from jax import lax
from jax.experimental import pallas as pl
from jax.experimental.pallas import tpu as pltpu

K = 4096
Q = 1024
N = 8192
W = 512
C = N // W
NSLOT = 3


def kernel(x, dy):
    mx = lax.axis_index("x")
    my = lax.axis_index("y")
    q_mine = 2 * mx + my
    q_send = 2 * (1 - mx) + my

    xs_m = lax.dynamic_slice(x, (0, q_mine * Q), (K, Q))
    xs_s = lax.dynamic_slice(x, (0, q_send * Q), (K, Q))
    xs_cat_t = jnp.concatenate(
        [xs_m.T.astype(jnp.bfloat16), xs_s.T.astype(jnp.bfloat16)], axis=0
    )

    def body(
        xs_cat_hbm, dy_ref, out_ref,
        xs_cat_vm,
        psend, pmine, recvx, red, recvy,
        cp_sems, send1, recv1, send2, recv2,
    ):
        i = pl.program_id(0)
        tx = lax.axis_index("x")
        ty = lax.axis_index("y")
        x_nbr = (1 - tx, ty)
        y_nbr = (tx, 1 - ty)
        barrier_sem = pltpu.get_barrier_semaphore()

        def mk1(slot):
            return pltpu.make_async_remote_copy(
                src_ref=psend.at[slot],
                dst_ref=recvx.at[slot],
                send_sem=send1.at[slot],
                recv_sem=recv1.at[slot],
                device_id=x_nbr,
                device_id_type=pl.DeviceIdType.MESH,
            )

        def mk2(slot):
            return pltpu.make_async_remote_copy(
                src_ref=red.at[slot],
                dst_ref=recvy.at[slot],
                send_sem=send2.at[slot],
                recv_sem=recv2.at[slot],
                device_id=y_nbr,
                device_id_type=pl.DeviceIdType.MESH,
            )

        def store_slot(ref, s, val):
            for k in range(NSLOT):
                @pl.when(s == k)
                def _(k=k):
                    ref[k, :, :] = val

        @pl.when(i == 0)
        def _():
            cp1 = pltpu.make_async_copy(xs_cat_hbm, xs_cat_vm,
                                        cp_sems.at[0])
            cp1.start()
            for nbr in (x_nbr, y_nbr):
                pl.semaphore_signal(
                    barrier_sem, inc=1,
                    device_id=nbr, device_id_type=pl.DeviceIdType.MESH,
                )
            pl.semaphore_wait(barrier_sem, 2)
            cp1.wait()

        @pl.when(i < C)
        def _():
            s = lax.rem(i, NSLOT)

            @pl.when(i >= NSLOT)
            def _():
                mk1(s).wait_send()

            dyb = dy_ref[...].astype(jnp.bfloat16)
            pq = jnp.dot(
                xs_cat_vm[...], dyb, preferred_element_type=jnp.float32
            ).astype(jnp.bfloat16)
            store_slot(psend, s, pq[Q:, :])
            mk1(s).start()
            store_slot(pmine, s, pq[:Q, :])

        @pl.when(jnp.logical_and(i >= 1, i <= C))
        def _():
            j = i - 1
            s = lax.rem(j, NSLOT)
            mk1(s).wait_recv()

            @pl.when(j >= NSLOT)
            def _():
                mk2(s).wait_send()

            rv = pmine[s] + recvx[s]
            store_slot(red, s, rv)
            mk2(s).start()

        @pl.when(i >= 2)
        def _():
            j = i - 2
            s = lax.rem(j, NSLOT)
            out_ref[pl.ds(ty * Q, Q), :] = red[s]
            mk2(s).wait_recv()
            out_ref[pl.ds((1 - ty) * Q, Q), :] = recvy[s]

        @pl.when(i == C + 1)
        def _():
            for k in range(NSLOT):
                mk1(k).wait_send()
                mk2(k).wait_send()

    out = pl.pallas_call(
        body,
        grid=(C + 2,),
        out_shape=jax.ShapeDtypeStruct((2 * Q, N), jnp.bfloat16),
        in_specs=[
            pl.BlockSpec(memory_space=pl.ANY),
            pl.BlockSpec((K, W), lambda i: (0, jnp.minimum(i, C - 1))),
        ],
        out_specs=pl.BlockSpec(
            (2 * Q, W), lambda i: (0, jnp.maximum(i - 2, 0))
        ),
        scratch_shapes=[
            pltpu.VMEM((2 * Q, K), jnp.bfloat16),
            pltpu.VMEM((NSLOT, Q, W), jnp.bfloat16),
            pltpu.VMEM((NSLOT, Q, W), jnp.bfloat16),
            pltpu.VMEM((NSLOT, Q, W), jnp.bfloat16),
            pltpu.VMEM((NSLOT, Q, W), jnp.bfloat16),
            pltpu.VMEM((NSLOT, Q, W), jnp.bfloat16),
            pltpu.SemaphoreType.DMA((1,)),
            pltpu.SemaphoreType.DMA((NSLOT,)),
            pltpu.SemaphoreType.DMA((NSLOT,)),
            pltpu.SemaphoreType.DMA((NSLOT,)),
            pltpu.SemaphoreType.DMA((NSLOT,)),
        ],
        compiler_params=pltpu.CompilerParams(
            collective_id=0, vmem_limit_bytes=62 * 1024 * 1024
        ),
    )(xs_cat_t, dy)
    return out


# device time: 267718 ns/iter; 3.0242x vs baseline; 1.0398x over previous
import jax
import jax.numpy as jnp
from jax import lax
from jax.experimental import pallas as pl
from jax.experimental.pallas import tpu as pltpu

K = 4096
Q = 1024
N = 8192
W = 512
C = N // W
NSLOT = 3


def kernel(x, dy):
    mx = lax.axis_index("x")
    my = lax.axis_index("y")
    q_mine = 2 * mx + my
    q_send = 2 * (1 - mx) + my

    xs_m = lax.dynamic_slice(x, (0, q_mine * Q), (K, Q))
    xs_s = lax.dynamic_slice(x, (0, q_send * Q), (K, Q))
    xs_cat = jnp.concatenate([xs_m, xs_s], axis=1).astype(jnp.bfloat16)

    def body(
        xs_cat_hbm, dy_ref, out_ref,
        xs_cat_vm,
        psend, pmine, recvx, red, recvy,
        cp_sems, send1, recv1, send2, recv2,
    ):
        i = pl.program_id(0)
        tx = lax.axis_index("x")
        ty = lax.axis_index("y")
        x_nbr = (1 - tx, ty)
        y_nbr = (tx, 1 - ty)
        barrier_sem = pltpu.get_barrier_semaphore()

        def mk1(slot):
            return pltpu.make_async_remote_copy(
                src_ref=psend.at[slot],
                dst_ref=recvx.at[slot],
                send_sem=send1.at[slot],
                recv_sem=recv1.at[slot],
                device_id=x_nbr,
                device_id_type=pl.DeviceIdType.MESH,
            )

        def mk2(slot):
            return pltpu.make_async_remote_copy(
                src_ref=red.at[slot],
                dst_ref=recvy.at[slot],
                send_sem=send2.at[slot],
                recv_sem=recv2.at[slot],
                device_id=y_nbr,
                device_id_type=pl.DeviceIdType.MESH,
            )

        def store_slot(ref, s, val):
            for k in range(NSLOT):
                @pl.when(s == k)
                def _(k=k):
                    ref[k, :, :] = val

        @pl.when(i == 0)
        def _():
            cp1 = pltpu.make_async_copy(xs_cat_hbm, xs_cat_vm,
                                        cp_sems.at[0])
            cp1.start()
            for nbr in (x_nbr, y_nbr):
                pl.semaphore_signal(
                    barrier_sem, inc=1,
                    device_id=nbr, device_id_type=pl.DeviceIdType.MESH,
                )
            pl.semaphore_wait(barrier_sem, 2)
            cp1.wait()

        @pl.when(i < C)
        def _():
            s = lax.rem(i, NSLOT)

            @pl.when(i >= NSLOT)
            def _():
                mk1(s).wait_send()

            dyb = dy_ref[...].astype(jnp.bfloat16)
            pq = lax.dot_general(
                xs_cat_vm[...], dyb,
                (((0,), (0,)), ((), ())),
                preferred_element_type=jnp.float32,
            ).astype(jnp.bfloat16)
            store_slot(psend, s, pq[Q:, :])
            mk1(s).start()
            store_slot(pmine, s, pq[:Q, :])

        @pl.when(jnp.logical_and(i >= 1, i <= C))
        def _():
            j = i - 1
            s = lax.rem(j, NSLOT)
            mk1(s).wait_recv()

            @pl.when(j >= NSLOT)
            def _():
                mk2(s).wait_send()

            rv = pmine[s] + recvx[s]
            store_slot(red, s, rv)
            mk2(s).start()

        @pl.when(i >= 2)
        def _():
            j = i - 2
            s = lax.rem(j, NSLOT)
            out_ref[pl.ds(ty * Q, Q), :] = red[s]
            mk2(s).wait_recv()
            out_ref[pl.ds((1 - ty) * Q, Q), :] = recvy[s]

        @pl.when(i == C + 1)
        def _():
            for k in range(NSLOT):
                mk1(k).wait_send()
                mk2(k).wait_send()

    out = pl.pallas_call(
        body,
        grid=(C + 2,),
        out_shape=jax.ShapeDtypeStruct((2 * Q, N), jnp.bfloat16),
        in_specs=[
            pl.BlockSpec(memory_space=pl.ANY),
            pl.BlockSpec((K, W), lambda i: (0, jnp.minimum(i, C - 1))),
        ],
        out_specs=pl.BlockSpec(
            (2 * Q, W), lambda i: (0, jnp.maximum(i - 2, 0))
        ),
        scratch_shapes=[
            pltpu.VMEM((K, 2 * Q), jnp.bfloat16),
            pltpu.VMEM((NSLOT, Q, W), jnp.bfloat16),
            pltpu.VMEM((NSLOT, Q, W), jnp.bfloat16),
            pltpu.VMEM((NSLOT, Q, W), jnp.bfloat16),
            pltpu.VMEM((NSLOT, Q, W), jnp.bfloat16),
            pltpu.VMEM((NSLOT, Q, W), jnp.bfloat16),
            pltpu.SemaphoreType.DMA((1,)),
            pltpu.SemaphoreType.DMA((NSLOT,)),
            pltpu.SemaphoreType.DMA((NSLOT,)),
            pltpu.SemaphoreType.DMA((NSLOT,)),
            pltpu.SemaphoreType.DMA((NSLOT,)),
        ],
        compiler_params=pltpu.CompilerParams(
            collective_id=0, vmem_limit_bytes=62 * 1024 * 1024
        ),
    )(xs_cat, dy)
    return out


# device time: 267520 ns/iter; 3.0264x vs baseline; 1.0007x over previous
import jax
import jax.numpy as jnp
from jax import lax
from jax.experimental import pallas as pl
from jax.experimental.pallas import tpu as pltpu

K = 4096
Q = 1024
N = 8192
W = 512
C = N // W
NSLOT = 3


def kernel(x, dy):
    mx = lax.axis_index("x")
    my = lax.axis_index("y")
    q_mine = 2 * mx + my
    q_send = 2 * (1 - mx) + my

    xs_m = lax.dynamic_slice(x, (0, q_mine * Q), (K, Q))
    xs_s = lax.dynamic_slice(x, (0, q_send * Q), (K, Q))
    xs_cat = jnp.concatenate([xs_m, xs_s], axis=1).astype(jnp.bfloat16)

    def body(
        xs_cat_hbm, dy_ref, out_ref,
        xs_cat_vm,
        psend, pmine, recvx, red, recvy,
        cp_sems, send1, recv1, send2, recv2,
    ):
        i = pl.program_id(0)
        tx = lax.axis_index("x")
        ty = lax.axis_index("y")
        x_nbr = (1 - tx, ty)
        y_nbr = (tx, 1 - ty)
        barrier_sem = pltpu.get_barrier_semaphore()

        def mk1(slot):
            return pltpu.make_async_remote_copy(
                src_ref=psend.at[slot],
                dst_ref=recvx.at[slot],
                send_sem=send1.at[slot],
                recv_sem=recv1.at[slot],
                device_id=x_nbr,
                device_id_type=pl.DeviceIdType.MESH,
            )

        def mk2(slot):
            return pltpu.make_async_remote_copy(
                src_ref=red.at[slot],
                dst_ref=recvy.at[slot],
                send_sem=send2.at[slot],
                recv_sem=recv2.at[slot],
                device_id=y_nbr,
                device_id_type=pl.DeviceIdType.MESH,
            )

        def store_slot(ref, s, val):
            for k in range(NSLOT):
                @pl.when(s == k)
                def _(k=k):
                    ref[k, :, :] = val

        @pl.when(i == 0)
        def _():
            cp1 = pltpu.make_async_copy(xs_cat_hbm, xs_cat_vm,
                                        cp_sems.at[0])
            cp1.start()
            for nbr in (x_nbr, y_nbr):
                pl.semaphore_signal(
                    barrier_sem, inc=1,
                    device_id=nbr, device_id_type=pl.DeviceIdType.MESH,
                )
            pl.semaphore_wait(barrier_sem, 2)
            cp1.wait()

        @pl.when(i < C)
        def _():
            s = lax.rem(i, NSLOT)

            @pl.when(i >= NSLOT)
            def _():
                mk1(s).wait_send()

            H = K // 2
            dn = (((0,), (0,)), ((), ()))
            dyb0 = dy_ref[:H, :].astype(jnp.bfloat16)
            acc = lax.dot_general(
                xs_cat_vm[:H, :], dyb0, dn,
                preferred_element_type=jnp.float32,
            )
            dyb1 = dy_ref[H:, :].astype(jnp.bfloat16)
            acc = acc + lax.dot_general(
                xs_cat_vm[H:, :], dyb1, dn,
                preferred_element_type=jnp.float32,
            )
            pq = acc.astype(jnp.bfloat16)
            store_slot(psend, s, pq[Q:, :])
            mk1(s).start()
            store_slot(pmine, s, pq[:Q, :])

        @pl.when(jnp.logical_and(i >= 1, i <= C))
        def _():
            j = i - 1
            s = lax.rem(j, NSLOT)
            mk1(s).wait_recv()

            @pl.when(j >= NSLOT)
            def _():
                mk2(s).wait_send()

            rv = pmine[s] + recvx[s]
            store_slot(red, s, rv)
            mk2(s).start()

        @pl.when(i >= 2)
        def _():
            j = i - 2
            s = lax.rem(j, NSLOT)
            out_ref[pl.ds(ty * Q, Q), :] = red[s]
            mk2(s).wait_recv()
            out_ref[pl.ds((1 - ty) * Q, Q), :] = recvy[s]

        @pl.when(i == C + 1)
        def _():
            for k in range(NSLOT):
                mk1(k).wait_send()
                mk2(k).wait_send()

    out = pl.pallas_call(
        body,
        grid=(C + 2,),
        out_shape=jax.ShapeDtypeStruct((2 * Q, N), jnp.bfloat16),
        in_specs=[
            pl.BlockSpec(memory_space=pl.ANY),
            pl.BlockSpec((K, W), lambda i: (0, jnp.minimum(i, C - 1))),
        ],
        out_specs=pl.BlockSpec(
            (2 * Q, W), lambda i: (0, jnp.maximum(i - 2, 0))
        ),
        scratch_shapes=[
            pltpu.VMEM((K, 2 * Q), jnp.bfloat16),
            pltpu.VMEM((NSLOT, Q, W), jnp.bfloat16),
            pltpu.VMEM((NSLOT, Q, W), jnp.bfloat16),
            pltpu.VMEM((NSLOT, Q, W), jnp.bfloat16),
            pltpu.VMEM((NSLOT, Q, W), jnp.bfloat16),
            pltpu.VMEM((NSLOT, Q, W), jnp.bfloat16),
            pltpu.SemaphoreType.DMA((1,)),
            pltpu.SemaphoreType.DMA((NSLOT,)),
            pltpu.SemaphoreType.DMA((NSLOT,)),
            pltpu.SemaphoreType.DMA((NSLOT,)),
            pltpu.SemaphoreType.DMA((NSLOT,)),
        ],
        compiler_params=pltpu.CompilerParams(
            collective_id=0, vmem_limit_bytes=62 * 1024 * 1024
        ),
    )(xs_cat, dy)
    return out
